# baseline (speedup 1.0000x reference)
import functools

import jax
import jax.numpy as jnp
from jax import lax
from jax.experimental import pallas as pl
from jax.experimental.pallas import tpu as pltpu
from jax.experimental.pallas import tpu_sc as plsc

NC = 2
NS = 16
L = 16
NW = NC * NS

BATCH = 16384
K = 16
PACK = 128 // K
B_PER_W = BATCH // NW
CHUNK = 128
N_CHUNK = B_PER_W // CHUNK
N_BLOCK = B_PER_W // L


def _mf_kernel(iu_hbm, ip_hbm, upack_hbm, ppack_hbm, w_hbm, b_hbm, out_hbm,
               idxu_v, idxp_v, rowid_v, rows_v, cols_u, cols_p, out_v,
               w_v, b_v, sem):
    wid = lax.axis_index("s") * NC + lax.axis_index("c")
    base_chunk = wid * N_CHUNK
    out_base = wid * B_PER_W

    pltpu.sync_copy(iu_hbm.at[pl.ds(base_chunk, N_CHUNK)], idxu_v)
    pltpu.sync_copy(ip_hbm.at[pl.ds(base_chunk, N_CHUNK)], idxp_v)
    pltpu.sync_copy(w_hbm, w_v)
    pltpu.sync_copy(b_hbm, b_v)

    wk_vecs = [w_v[f, :] for f in range(K)]
    bias = b_v[...]
    iota = lax.iota(jnp.int32, L)
    col_f = [jnp.full((L,), f, jnp.int32) for f in range(K)]

    def stage(idx2d, tab_hbm, cols):
        for c in range(N_CHUNK):
            for j in range(CHUNK // L):
                v = idx2d[c, pl.ds(j * L, L)]
                rowid_v[pl.ds(c * CHUNK + j * L, L)] = v >> 3
        copies = []
        for c in range(N_CHUNK):
            copies.append(pltpu.async_copy(
                tab_hbm.at[rowid_v.at[pl.ds(c * CHUNK, CHUNK)]],
                rows_v.at[pl.ds(c * CHUNK, CHUNK)], sem))
        for cp in copies:
            cp.wait()
        for blk in range(N_BLOCK):
            c, j = divmod(blk, CHUNK // L)
            v = idx2d[c, pl.ds(j * L, L)]
            row_idx = blk * L + iota
            off = (v & 7) << 4
            for f in range(K):
                val = plsc.load_gather(rows_v, [row_idx, off + col_f[f]])
                cols[pl.ds(f * B_PER_W + blk * L, L)] = val

    stage(idxu_v, upack_hbm, cols_u)
    stage(idxp_v, ppack_hbm, cols_p)

    def block(blk, _):
        acc = bias
        for f in range(K):
            uu = cols_u[pl.ds(f * B_PER_W + blk * L, L)]
            pp = cols_p[pl.ds(f * B_PER_W + blk * L, L)]
            acc = acc + (uu * pp) * wk_vecs[f]
        out_v[pl.ds(blk * L, L)] = acc
        return 0

    lax.fori_loop(0, N_BLOCK, block, 0)

    pltpu.sync_copy(out_v, out_hbm.at[pl.ds(out_base, B_PER_W)])


@jax.jit
def _mf(iu, ip, upack, ppack, w_bcast, b_vec):
    run = pl.kernel(
        _mf_kernel,
        out_type=jax.ShapeDtypeStruct((BATCH,), jnp.float32),
        mesh=plsc.VectorSubcoreMesh(core_axis_name="c", subcore_axis_name="s",
                                    num_cores=NC, num_subcores=NS),
        compiler_params=pltpu.CompilerParams(needs_layout_passes=False),
        scratch_types=[
            pltpu.VMEM((N_CHUNK, CHUNK), jnp.int32),
            pltpu.VMEM((N_CHUNK, CHUNK), jnp.int32),
            pltpu.VMEM((B_PER_W,), jnp.int32),
            pltpu.VMEM((B_PER_W, 128), jnp.float32),
            pltpu.VMEM((K * B_PER_W,), jnp.float32),
            pltpu.VMEM((K * B_PER_W,), jnp.float32),
            pltpu.VMEM((B_PER_W,), jnp.float32),
            pltpu.VMEM((K, L), jnp.float32),
            pltpu.VMEM((L,), jnp.float32),
            pltpu.SemaphoreType.DMA,
        ],
    )
    return run(iu, ip, upack, ppack, w_bcast, b_vec)


def kernel(input_user, input_prob, user_emb, prob_emb, dense_w, dense_b):
    iu = input_user.reshape(NW * N_CHUNK, CHUNK)
    ip = input_prob.reshape(NW * N_CHUNK, CHUNK)
    upack = user_emb.reshape(user_emb.shape[0] // PACK, 128)
    ppack = prob_emb.reshape(prob_emb.shape[0] // PACK, 128)
    w_bcast = jnp.broadcast_to(dense_w.reshape(K, 1), (K, L))
    b_vec = jnp.broadcast_to(dense_b, (L,))
    out = _mf(iu, ip, upack, ppack, w_bcast, b_vec)
    return out.reshape(BATCH, 1)
